# Initial kernel scaffold; baseline (speedup 1.0000x reference)
import jax
import jax.numpy as jnp
from jax.experimental import pallas as pl


def kernel(neighbor_index, atoms, angle_weights):
    raise NotImplementedError("write your pallas kernel here")


# speedup vs baseline: 201.0652x; 201.0652x over previous
import functools

import jax
import jax.numpy as jnp
from jax import lax
from jax.experimental import pallas as pl
from jax.experimental.pallas import tpu as pltpu
from jax.experimental.pallas import tpu_sc as plsc

_BS = 16
_ATOM = 10000
_NEI = 65
_KN = 16
_HALF = _ATOM // 2
_CHUNK = 200
_NCHUNK = _HALF // _CHUNK
_EPS2 = 1e-24


def _rsqrt(x):
    i = plsc.bitcast(x, jnp.int32)
    y = plsc.bitcast(jnp.int32(0x5F3759DF) - (i >> 1), jnp.float32)
    y = y * (1.5 - 0.5 * x * y * y)
    y = y * (1.5 - 0.5 * x * y * y)
    return y


def _body(idx_hbm, pos_hbm, out_hbm, pos_v, idx_v, out_v):
    b = lax.axis_index("s")
    h = lax.axis_index("c")

    pltpu.sync_copy(pos_hbm.at[b], pos_v)

    def chunk_body(chunk, _):
        astart = pl.multiple_of(h * _HALF + chunk * _CHUNK, 8)
        pltpu.sync_copy(idx_hbm.at[b, pl.ds(astart, _CHUNK)], idx_v)

        def atom_body(i, _):
            gav = jnp.full((_KN,), astart + i, jnp.int32)
            cx = plsc.load_gather(pos_v, [gav])
            cy = plsc.load_gather(pos_v, [gav + _ATOM])
            cz = plsc.load_gather(pos_v, [gav + 2 * _ATOM])
            row64 = idx_v[i, pl.ds(64, 16)]
            i0v = jnp.full((_KN,), row64[0], jnp.int32)
            d0x = plsc.load_gather(pos_v, [i0v]) - cx
            d0y = plsc.load_gather(pos_v, [i0v + _ATOM]) - cy
            d0z = plsc.load_gather(pos_v, [i0v + 2 * _ATOM]) - cz
            s0 = d0x * d0x + d0y * d0y + d0z * d0z
            r0 = _rsqrt(jnp.maximum(s0, _EPS2))
            u0x = d0x * r0
            u0y = d0y * r0
            u0z = d0z * r0

            acc = jnp.zeros((_KN,), jnp.float32)
            for g in range(4):
                idxg = idx_v[i, pl.ds(16 * g, 16)]
                gx = plsc.load_gather(pos_v, [idxg])
                gy = plsc.load_gather(pos_v, [idxg + _ATOM])
                gz = plsc.load_gather(pos_v, [idxg + 2 * _ATOM])
                dx = gx - cx
                dy = gy - cy
                dz = gz - cz
                q = dx * u0x + dy * u0y + dz * u0z
                ss = dx * dx + dy * dy + dz * dz
                r = _rsqrt(jnp.maximum(ss, _EPS2))
                acc = acc + jnp.maximum(q, 0.0) * r
            out_v[i] = acc
            return 0

        lax.fori_loop(0, _CHUNK, atom_body, 0)
        pltpu.sync_copy(out_v, out_hbm.at[b, pl.ds(astart, _CHUNK)])
        return 0

    lax.fori_loop(0, _NCHUNK, chunk_body, 0)


@jax.jit
def _atom_conv(idx80, pos_flat):
    mesh = plsc.VectorSubcoreMesh(core_axis_name="c", subcore_axis_name="s")
    fn = pl.kernel(
        _body,
        out_type=jax.ShapeDtypeStruct((_BS, _ATOM, _KN), jnp.float32),
        mesh=mesh,
        scratch_types=[
            pltpu.VMEM((3 * _ATOM,), jnp.float32),
            pltpu.VMEM((_CHUNK, 80), jnp.int32),
            pltpu.VMEM((_CHUNK, _KN), jnp.float32),
        ],
        compiler_params=pltpu.CompilerParams(needs_layout_passes=False),
    )
    return fn(idx80, pos_flat)


def kernel(neighbor_index, atoms, angle_weights):
    del angle_weights
    idx80 = jnp.concatenate(
        [
            neighbor_index[:, :, 1:],
            neighbor_index[:, :, :1],
            jnp.zeros((_BS, _ATOM, 15), jnp.int32),
        ],
        axis=-1,
    )
    pos_flat = atoms.transpose(0, 2, 1).reshape(_BS, 3 * _ATOM)
    return _atom_conv(idx80, pos_flat)


# speedup vs baseline: 239.5807x; 1.1916x over previous
import functools

import jax
import jax.numpy as jnp
from jax import lax
from jax.experimental import pallas as pl
from jax.experimental.pallas import tpu as pltpu
from jax.experimental.pallas import tpu_sc as plsc

_BS = 16
_ATOM = 10000
_NEI = 65
_KN = 16
_HALF = _ATOM // 2
_CHUNK = 200
_NCHUNK = _HALF // _CHUNK
_EPS2 = 1e-24


def _rsqrt(x):
    i = plsc.bitcast(x, jnp.int32)
    y = plsc.bitcast(jnp.int32(0x5F3759DF) - (i >> 1), jnp.float32)
    y = y * (1.5 - 0.5 * x * y * y)
    return y


def _body(idx_hbm, pos_hbm, out_hbm, pos_v, idx_v, out_v):
    b = lax.axis_index("s")
    h = lax.axis_index("c")

    pltpu.sync_copy(pos_hbm.at[b], pos_v)

    def chunk_body(chunk, _):
        astart = pl.multiple_of(h * _HALF + chunk * _CHUNK, 8)
        pltpu.sync_copy(idx_hbm.at[b, pl.ds(astart, _CHUNK)], idx_v)

        @plsc.parallel_loop(0, _CHUNK, unroll=4)
        def _(i):
            gav = jnp.full((_KN,), astart + i, jnp.int32)
            cx = plsc.load_gather(pos_v, [gav])
            cy = plsc.load_gather(pos_v, [gav + _ATOM])
            cz = plsc.load_gather(pos_v, [gav + 2 * _ATOM])
            row64 = idx_v[i, pl.ds(64, 16)]
            i0v = jnp.full((_KN,), row64[0], jnp.int32)
            d0x = plsc.load_gather(pos_v, [i0v]) - cx
            d0y = plsc.load_gather(pos_v, [i0v + _ATOM]) - cy
            d0z = plsc.load_gather(pos_v, [i0v + 2 * _ATOM]) - cz
            s0 = d0x * d0x + d0y * d0y + d0z * d0z
            r0 = _rsqrt(jnp.maximum(s0, _EPS2))

            acc = jnp.zeros((_KN,), jnp.float32)
            for g in range(4):
                idxg = idx_v[i, pl.ds(16 * g, 16)]
                gx = plsc.load_gather(pos_v, [idxg])
                gy = plsc.load_gather(pos_v, [idxg + _ATOM])
                gz = plsc.load_gather(pos_v, [idxg + 2 * _ATOM])
                dx = gx - cx
                dy = gy - cy
                dz = gz - cz
                q = dx * d0x + dy * d0y + dz * d0z
                ss = dx * dx + dy * dy + dz * dz
                r = _rsqrt(jnp.maximum(ss, _EPS2))
                acc = acc + jnp.maximum(q, 0.0) * r
            out_v[i] = acc * r0
        pltpu.sync_copy(out_v, out_hbm.at[b, pl.ds(astart, _CHUNK)])
        return 0

    lax.fori_loop(0, _NCHUNK, chunk_body, 0)


@jax.jit
def _atom_conv(idx80, pos_flat):
    mesh = plsc.VectorSubcoreMesh(core_axis_name="c", subcore_axis_name="s")
    fn = pl.kernel(
        _body,
        out_type=jax.ShapeDtypeStruct((_BS, _ATOM, _KN), jnp.float32),
        mesh=mesh,
        scratch_types=[
            pltpu.VMEM((3 * _ATOM,), jnp.float32),
            pltpu.VMEM((_CHUNK, 80), jnp.int32),
            pltpu.VMEM((_CHUNK, _KN), jnp.float32),
        ],
        compiler_params=pltpu.CompilerParams(needs_layout_passes=False),
    )
    return fn(idx80, pos_flat)


def kernel(neighbor_index, atoms, angle_weights):
    del angle_weights
    idx80 = jnp.concatenate(
        [
            neighbor_index[:, :, 1:],
            neighbor_index[:, :, :1],
            jnp.zeros((_BS, _ATOM, 15), jnp.int32),
        ],
        axis=-1,
    )
    pos_flat = atoms.transpose(0, 2, 1).reshape(_BS, 3 * _ATOM)
    return _atom_conv(idx80, pos_flat)


# speedup vs baseline: 296.0793x; 1.2358x over previous
import functools

import jax
import jax.numpy as jnp
from jax import lax
from jax.experimental import pallas as pl
from jax.experimental.pallas import tpu as pltpu
from jax.experimental.pallas import tpu_sc as plsc

_BS = 16
_ATOM = 10000
_NEI = 65
_KN = 16
_HALF = _ATOM // 2
_CHUNK = 200
_NCHUNK = _HALF // _CHUNK
_EPS2 = 1e-24


def _rsqrt(x):
    i = plsc.bitcast(x, jnp.int32)
    y = plsc.bitcast(jnp.int32(0x5F3759DF) - (i >> 1), jnp.float32)
    y = y * (1.5 - 0.5 * x * y * y)
    return y


def _body(idx_hbm, pos_hbm, out_hbm, pos_v, idx_v, out_v):
    b = lax.axis_index("s")
    h = lax.axis_index("c")

    pltpu.sync_copy(pos_hbm.at[b], pos_v)

    def chunk_body(chunk, _):
        astart = pl.multiple_of(h * _HALF + chunk * _CHUNK, 8)
        pltpu.sync_copy(idx_hbm.at[b, pl.ds(astart, _CHUNK)], idx_v)

        @plsc.parallel_loop(0, _CHUNK, unroll=8)
        def _(i):
            gav = jnp.full((_KN,), astart + i, jnp.int32)
            cx = plsc.load_gather(pos_v, [gav])
            cy = plsc.load_gather(pos_v, [gav + _ATOM])
            cz = plsc.load_gather(pos_v, [gav + 2 * _ATOM])
            row0 = idx_v[i, pl.ds(0, 16)]
            i0v = jnp.full((_KN,), row0[0], jnp.int32)
            d0x = plsc.load_gather(pos_v, [i0v]) - cx
            d0y = plsc.load_gather(pos_v, [i0v + _ATOM]) - cy
            d0z = plsc.load_gather(pos_v, [i0v + 2 * _ATOM]) - cz
            s0 = d0x * d0x + d0y * d0y + d0z * d0z
            r0 = _rsqrt(jnp.maximum(s0, _EPS2))

            acc = jnp.zeros((_KN,), jnp.float32)
            for g in range(4):
                idxg = idx_v[i, pl.ds(1 + 16 * g, 16)]
                gx = plsc.load_gather(pos_v, [idxg])
                gy = plsc.load_gather(pos_v, [idxg + _ATOM])
                gz = plsc.load_gather(pos_v, [idxg + 2 * _ATOM])
                dx = gx - cx
                dy = gy - cy
                dz = gz - cz
                q = dx * d0x + dy * d0y + dz * d0z
                ss = dx * dx + dy * dy + dz * dz
                r = _rsqrt(jnp.maximum(ss, _EPS2))
                acc = acc + jnp.maximum(q, 0.0) * r
            out_v[i] = acc * r0
        pltpu.sync_copy(out_v, out_hbm.at[b, pl.ds(astart, _CHUNK)])
        return 0

    lax.fori_loop(0, _NCHUNK, chunk_body, 0)


@jax.jit
def _atom_conv(neighbor_index, pos_flat):
    mesh = plsc.VectorSubcoreMesh(core_axis_name="c", subcore_axis_name="s")
    fn = pl.kernel(
        _body,
        out_type=jax.ShapeDtypeStruct((_BS, _ATOM, _KN), jnp.float32),
        mesh=mesh,
        scratch_types=[
            pltpu.VMEM((3 * _ATOM,), jnp.float32),
            pltpu.VMEM((_CHUNK, _NEI), jnp.int32),
            pltpu.VMEM((_CHUNK, _KN), jnp.float32),
        ],
        compiler_params=pltpu.CompilerParams(needs_layout_passes=False),
    )
    return fn(neighbor_index, pos_flat)


def kernel(neighbor_index, atoms, angle_weights):
    del angle_weights
    pos_flat = atoms.transpose(0, 2, 1).reshape(_BS, 3 * _ATOM)
    return _atom_conv(neighbor_index, pos_flat)
